# baseline (speedup 1.0000x reference)
import functools

import jax
import jax.numpy as jnp
from jax import lax
from jax.experimental import pallas as pl
from jax.experimental.pallas import tpu as pltpu
from jax.experimental.pallas import tpu_sc as plsc

NC = 2
NS = 16
NW = NC * NS
L = 16


def _make_sc_kernel(B, V, D):
    OUT = 3 * D + 3
    FP = 392
    FT = FP // 8
    CT = B // 128
    BPW = B // NW
    CTW = BPW // 128
    FTC = 7
    NCHUNK = FT // FTC
    GRP = BPW // L

    mesh = plsc.VectorSubcoreMesh(core_axis_name="c", subcore_axis_name="s")

    @functools.partial(
        pl.kernel,
        mesh=mesh,
        compiler_params=pltpu.CompilerParams(
            use_tc_tiling_on_sc=False, needs_layout_passes=False,
            disable_bounds_checks=True, skip_device_barrier=True),
        out_type=jax.ShapeDtypeStruct((FT, CT, 8, 128), jnp.float32),
        scratch_types=[
            pltpu.VMEM((BPW,), jnp.int32),
            pltpu.VMEM((BPW,), jnp.float32),
            pltpu.VMEM((BPW,), jnp.float32),
            pltpu.VMEM((BPW,), jnp.float32),
            pltpu.VMEM((BPW,), jnp.int32),
            pltpu.VMEM((BPW,), jnp.int32),
            pltpu.VMEM((BPW, D), jnp.float32),
            pltpu.VMEM((FTC, CTW, 8, 131), jnp.float32),
            pltpu.VMEM((FTC, CTW, 8, 131), jnp.float32),
            pltpu.SemaphoreType.DMA,
            pltpu.SemaphoreType.DMA,
            pltpu.SemaphoreType.DMA,
            pltpu.SemaphoreType.DMA,
        ],
    )
    def sc_kernel(uid_hbm, age_hbm, hr_hbm, mo_hbm, occ_hbm, gen_hbm,
                  table_hbm, out_hbm,
                  idx_v, age_v, hr_v, mo_v, occ_v, gen_v, emb_v,
                  bufA, bufB, sg, s_in, soA, soB):
        wid = lax.axis_index("s") * NC + lax.axis_index("c")
        base = wid * BPW
        ct0 = wid * CTW
        bufs = (bufA, bufB)
        sos = (soA, soB)

        pltpu.sync_copy(uid_hbm.at[pl.ds(base, BPW)], idx_v)
        gather = pltpu.async_copy(table_hbm.at[idx_v], emb_v, sg)
        stage = [
            pltpu.async_copy(occ_hbm.at[pl.ds(base, BPW)], occ_v, s_in),
            pltpu.async_copy(gen_hbm.at[pl.ds(base, BPW)], gen_v, s_in),
            pltpu.async_copy(age_hbm.at[pl.ds(base, BPW)], age_v, s_in),
            pltpu.async_copy(hr_hbm.at[pl.ds(base, BPW)], hr_v, s_in),
            pltpu.async_copy(mo_hbm.at[pl.ds(base, BPW)], mo_v, s_in),
        ]

        iota = lax.iota(jnp.int32, L)
        zeros = jnp.zeros((L,), jnp.float32)
        ones = jnp.ones((L,), jnp.float32)

        def zero_fill(buf, ftl_lo):
            @plsc.parallel_loop(0, (FTC - ftl_lo) * CTW * 8, 1, unroll=2)
            def _zf(i):
                ftl = ftl_lo + (i >> 5)
                ct = (i >> 3) & 3
                fr = i & 7
                for k in range(8):
                    buf[ftl, ct, fr, pl.ds(k * L, L)] = zeros
        def scatter_vals(buf, c, vals):
            fb = c * FTC * 8
            fe = fb + FTC * 8

            def _ones(g, _):
                col = (g & 7) * L + iota
                ct = g >> 3
                src = pl.ds(ct * 128 + (g & 7) * L, L)
                fo = D + 3 + occ_v[src]
                fg = 2 * D + 3 + gen_v[src]
                ctv = jnp.full((L,), ct, jnp.int32)
                if fb < 2 * D + 3:
                    plsc.store_scatter(
                        buf, [(fo - fb) >> 3, ctv, fo & 7, col], vals,
                        mask=(fo >= fb) & (fo < fe))
                if fe > 2 * D + 3:
                    plsc.store_scatter(
                        buf, [(fg - fb) >> 3, ctv, fg & 7, col], vals,
                        mask=(fg >= fb) & (fg < fe))
                return 0
            lax.fori_loop(0, GRP, _ones, 0)

        def copy_out(buf, c, so):
            return pltpu.async_copy(
                buf.at[:, :, :, pl.ds(0, 128)],
                out_hbm.at[pl.ds(c * FTC, FTC), pl.ds(ct0, CTW)], so)

        stage[0].wait()
        stage[1].wait()
        last = [None, None]
        prevc = [None, None]
        for i, c in enumerate((3, 4, 5, 6)):
            b = i % 2
            if last[b] is not None:
                last[b].wait()
                scatter_vals(bufs[b], prevc[b], zeros)
            else:
                zero_fill(bufs[b], 0)
            scatter_vals(bufs[b], c, ones)
            prevc[b] = c
            last[b] = copy_out(bufs[b], c, sos[b])

        stage[2].wait()
        stage[3].wait()
        stage[4].wait()
        gather.wait()

        for i, c in enumerate((0, 1, 2)):
            b = i % 2
            last[b].wait()
            buf = bufs[b]
            fb = c * FTC * 8
            n_emb = min(fb + FTC * 8, D) - fb

            f_hi = min(fb + FTC * 8, D)
            groups = []
            for f0 in range(0, D, L):
                if f0 + L > fb and f0 < f_hi:
                    fvec = f0 + iota
                    full = f0 >= fb and f0 + L <= f_hi
                    groups.append((
                        f0,
                        (fvec - fb) >> 3,
                        fvec & 7,
                        None if full else (fvec >= fb) & (fvec < f_hi),
                    ))

            @plsc.parallel_loop(0, BPW, 1, unroll=4)
            def _embf(bb, buf=buf, groups=groups):
                ctv = jnp.full((L,), bb >> 7, jnp.int32)
                colv = jnp.full((L,), bb & 127, jnp.int32)
                for f0, ftlv, frv, m in groups:
                    vals = emb_v[bb, pl.ds(f0, L)]
                    plsc.store_scatter(buf, [ftlv, ctv, frv, colv], vals,
                                       mask=m)

            if c == 2:
                def _scal(ct, _, buf=buf):
                    for k in range(8):
                        sl = pl.ds(ct * 128 + k * L, L)
                        buf[2, ct, 0, pl.ds(k * L, L)] = age_v[sl]
                        buf[2, ct, 1, pl.ds(k * L, L)] = hr_v[sl]
                        buf[2, ct, 2, pl.ds(k * L, L)] = mo_v[sl]
                    return 0
                lax.fori_loop(0, CTW, _scal, 0)
                zero_fill(buf, 3)

                def _z2(i2, _, buf=buf):
                    ct = i2 >> 3
                    k = i2 & 7
                    for fr in range(3, 8):
                        buf[2, ct, fr, pl.ds(k * L, L)] = zeros
                    return 0
                lax.fori_loop(0, CTW * 8, _z2, 0)
                scatter_vals(buf, c, ones)

            last[b] = copy_out(buf, c, sos[b])

        last[0].wait()
        last[1].wait()

    return sc_kernel


def kernel(user_id, age, hr_wk, month, occupation, gender, user_table):
    B = user_id.shape[0]
    V, D = user_table.shape
    OUT = 3 * D + 3
    FP = OUT + (-OUT) % 8
    sc = _make_sc_kernel(B, V, D)
    t = sc(
        user_id.astype(jnp.int32),
        age.reshape(B),
        hr_wk.reshape(B),
        month.reshape(B),
        occupation.astype(jnp.int32),
        gender.astype(jnp.int32),
        user_table,
    )
    t = t.transpose(0, 2, 1, 3).reshape(FP, B).T
    return t[:, :OUT]
